# baseline (speedup 1.0000x reference)
import functools
import jax
import jax.numpy as jnp
from jax import lax
from jax.experimental import pallas as pl
from jax.experimental.pallas import tpu as pltpu
from jax.experimental.pallas import tpu_sc as plsc

D_MODEL = 64
SCALE = 8.0
NUM_SUBCORES = 16
CHUNK = 128
NBUF = 8
OUT_W = 128


def _make_gather(batch: int):
    assert batch % (NUM_SUBCORES * CHUNK) == 0
    b_per_w = batch // NUM_SUBCORES
    n_chunks = b_per_w // CHUNK
    mesh = plsc.VectorSubcoreMesh(core_axis_name="c", subcore_axis_name="s",
                                  num_cores=1)

    @functools.partial(
        pl.kernel,
        mesh=mesh,
        out_type=jax.ShapeDtypeStruct((batch, OUT_W), jnp.float32),
        scratch_types=[
            pltpu.VMEM((n_chunks, CHUNK), jnp.int32),
            pltpu.VMEM((NBUF, CHUNK, D_MODEL), jnp.float32),
            pltpu.SemaphoreType.DMA((NBUF,)),
            pltpu.SemaphoreType.DMA((NBUF,)),
        ],
        compiler_params=pltpu.CompilerParams(use_tc_tiling_on_sc=False),
    )
    def gather_kernel(x_hbm, table_hbm, out_hbm, idx_v, rows_v, gsem, osem):
        wid = lax.axis_index("s")
        base = wid * b_per_w
        pltpu.sync_copy(x_hbm.at[wid], idx_v)

        def fire_gather(f, bf):
            pltpu.async_copy(table_hbm.at[idx_v.at[f]], rows_v.at[bf],
                             gsem.at[bf])

        def wait_gather(b):
            pltpu.make_async_copy(table_hbm.at[idx_v.at[0]], rows_v.at[b],
                                  gsem.at[b]).wait()

        def fire_write(j, b):
            pltpu.async_copy(
                rows_v.at[b],
                out_hbm.at[pl.ds(base + j * CHUNK, CHUNK), pl.ds(0, D_MODEL)],
                osem.at[b])

        def wait_write(b):
            pltpu.make_async_copy(
                rows_v.at[b],
                out_hbm.at[pl.ds(base, CHUNK), pl.ds(0, D_MODEL)],
                osem.at[b]).wait()

        for i in range(NBUF - 2):
            fire_gather(i, i)

        def chunk_body(j, carry):
            b = lax.rem(j, NBUF)
            f = j + NBUF - 2
            bf = lax.rem(f, NBUF)

            @pl.when(f < n_chunks)
            def _():
                @pl.when(f >= NBUF)
                def _():
                    wait_write(bf)
                fire_gather(f, bf)

            wait_gather(b)

            @plsc.parallel_loop(0, CHUNK, 1, unroll=8)
            def _(i):
                for d in range(D_MODEL // 16):
                    sl = pl.ds(d * 16, 16)
                    rows_v[b, i, sl] = rows_v[b, i, sl] * SCALE

            fire_write(j, b)
            return carry

        lax.fori_loop(0, n_chunks, chunk_body, 0)

        for c in range(n_chunks - NBUF, n_chunks):
            wait_write(c % NBUF)

    return gather_kernel


def kernel(x, table):
    batch, hist = x.shape
    total = batch * hist
    half = total // 2
    xf = x.reshape(2, NUM_SUBCORES, half // (NUM_SUBCORES * CHUNK), CHUNK)
    gather = _make_gather(half)
    lin0 = gather(xf[0], table)
    lin1 = gather(xf[1], table)
    lin = jnp.concatenate([lin0, lin1], axis=0)
    return lin.reshape(batch, hist, OUT_W)[:, :, :D_MODEL]


# speedup vs baseline: 1.3006x; 1.3006x over previous
import functools
import jax
import jax.numpy as jnp
from jax import lax
from jax.experimental import pallas as pl
from jax.experimental.pallas import tpu as pltpu
from jax.experimental.pallas import tpu_sc as plsc

D_MODEL = 64
SCALE = 8.0
NUM_CORES = 2
NUM_SUBCORES = 16
NUM_WORKERS = NUM_CORES * NUM_SUBCORES
SPLITS = ((0, 104), (104, 96))
NBUF = 6
OUT_W = 128


def _make_gather(batch: int, hist: int):
    rows_per_w = batch // NUM_WORKERS
    total = batch * hist
    mesh = plsc.VectorSubcoreMesh(core_axis_name="c", subcore_axis_name="s")

    @functools.partial(
        pl.kernel,
        mesh=mesh,
        out_type=jax.ShapeDtypeStruct((total, OUT_W), jnp.float32),
        scratch_types=[
            pltpu.VMEM((rows_per_w, hist), jnp.int32),
            pltpu.VMEM((NBUF, hist, D_MODEL), jnp.float32),
            pltpu.SemaphoreType.DMA((NBUF,)),
            pltpu.SemaphoreType.DMA((NBUF,)),
        ],
        compiler_params=pltpu.CompilerParams(use_tc_tiling_on_sc=False),
    )
    def gather_kernel(x_hbm, table_hbm, out_hbm, idx_v, rows_v, gsem, osem):
        wid = lax.axis_index("s") * NUM_CORES + lax.axis_index("c")
        base = wid * rows_per_w * hist
        pltpu.sync_copy(x_hbm.at[wid], idx_v)

        def fire_gather(r, b):
            for off, n in SPLITS:
                pltpu.async_copy(
                    table_hbm.at[idx_v.at[r, pl.ds(off, n)]],
                    rows_v.at[b, pl.ds(off, n)],
                    gsem.at[b])

        def wait_gather(b):
            for off, n in SPLITS:
                pltpu.make_async_copy(
                    table_hbm.at[idx_v.at[0, pl.ds(0, n)]],
                    rows_v.at[b, pl.ds(off, n)],
                    gsem.at[b]).wait()

        def fire_write(r, b):
            pltpu.async_copy(
                rows_v.at[b],
                out_hbm.at[pl.ds(base + r * hist, hist), pl.ds(0, D_MODEL)],
                osem.at[b])

        def wait_write(b):
            pltpu.make_async_copy(
                rows_v.at[b],
                out_hbm.at[pl.ds(base, hist), pl.ds(0, D_MODEL)],
                osem.at[b]).wait()

        for i in range(NBUF - 2):
            fire_gather(i, i)

        def row_step(r, carry):
            b = lax.rem(r, NBUF)
            f = r + NBUF - 2
            bf = lax.rem(f, NBUF)

            @pl.when(f < rows_per_w)
            def _():
                @pl.when(f >= NBUF)
                def _():
                    wait_write(bf)
                fire_gather(f, bf)

            wait_gather(b)

            @plsc.parallel_loop(0, hist, 1, unroll=8)
            def _(i):
                for d in range(D_MODEL // 16):
                    sl = pl.ds(d * 16, 16)
                    rows_v[b, i, sl] = rows_v[b, i, sl] * SCALE

            fire_write(r, b)
            return carry

        lax.fori_loop(0, rows_per_w, row_step, 0)

        for c in range(rows_per_w - NBUF, rows_per_w):
            wait_write(c % NBUF)

    return gather_kernel


def kernel(x, table):
    batch, hist = x.shape
    xf = x.reshape(NUM_WORKERS, batch // NUM_WORKERS, hist)
    lin = _make_gather(batch, hist)(xf, table)
    return lin.reshape(batch, hist, OUT_W)[:, :, :D_MODEL]
